# baseline (speedup 1.0000x reference)
import functools

import jax
import jax.numpy as jnp
from jax import lax
from jax.experimental import pallas as pl
from jax.experimental.pallas import tpu as pltpu
from jax.experimental.pallas import tpu_sc as plsc


@functools.lru_cache(maxsize=None)
def _build_sc_kernel(B, H, W, C, P):
    info = plsc.get_sparse_core_info()
    NC, NS, L = info.num_cores, info.num_subcores, info.num_lanes
    NW = NC * NS
    R = B * P
    rpw = R // NW
    assert R % NW == 0 and P % rpw == 0 and rpw % 128 == 0 and C % L == 0
    n_chunks = rpw // 128
    HW = H * W

    mesh = plsc.VectorSubcoreMesh(core_axis_name="c", subcore_axis_name="s")

    @functools.partial(
        pl.kernel,
        mesh=mesh,
        out_type=jax.ShapeDtypeStruct((R, C), jnp.float32),
        compiler_params=pltpu.CompilerParams(
            needs_layout_passes=False, use_tc_tiling_on_sc=False
        ),
        scratch_types=[
            pltpu.VMEM((rpw,), jnp.int32),
            pltpu.VMEM((rpw,), jnp.int32),
            pltpu.VMEM((rpw,), jnp.float32),
            pltpu.VMEM((n_chunks, 128), jnp.int32),
            pltpu.VMEM((rpw, C), jnp.float32),
            pltpu.SemaphoreType.DMA,
        ],
    )
    def sc_kernel(table, yy, xx, mm, out, y_v, x_v, m_v, idx_v, rows_v, sem):
        wid = lax.axis_index("s") * NC + lax.axis_index("c")
        base = wid * rpw
        pltpu.sync_copy(yy.at[pl.ds(base, rpw)], y_v)
        pltpu.sync_copy(xx.at[pl.ds(base, rpw)], x_v)
        pltpu.sync_copy(mm.at[pl.ds(base, rpw)], m_v)

        b_off = (base // P) * HW
        per_row = 128 // L
        for k in range(rpw // L):
            yv = y_v[pl.ds(k * L, L)]
            xv = x_v[pl.ds(k * L, L)]
            idx_v[k // per_row, pl.ds((k % per_row) * L, L)] = (
                yv * W + xv + b_off
            )

        copies = [
            pltpu.async_copy(
                table.at[idx_v.at[j]],
                rows_v.at[pl.ds(j * 128, 128)],
                sem,
            )
            for j in range(n_chunks)
        ]
        for c in copies:
            c.wait()

        def mul_body(r, carry):
            m16 = plsc.load_gather(m_v, [lax.broadcast(r, (L,))])
            for d in range(C // L):
                rows_v[r, pl.ds(d * L, L)] = rows_v[r, pl.ds(d * L, L)] * m16
            return carry

        lax.fori_loop(0, rpw, mul_body, 0)

        pltpu.sync_copy(rows_v, out.at[pl.ds(base, rpw)])

    return sc_kernel


def kernel(embeddings, coords, mask):
    B, H, W, C = embeddings.shape
    P = coords.shape[1]
    table = embeddings.reshape(B * H * W, C)
    c32 = coords.astype(jnp.int32)
    yy = c32[..., 0].reshape(-1)
    xx = c32[..., 1].reshape(-1)
    mm = mask.reshape(-1)
    out = _build_sc_kernel(B, H, W, C, P)(table, yy, xx, mm)
    return out.reshape(B, P, C)


# speedup vs baseline: 1.6025x; 1.6025x over previous
import functools

import jax
import jax.numpy as jnp
from jax import lax
from jax.experimental import pallas as pl
from jax.experimental.pallas import tpu as pltpu
from jax.experimental.pallas import tpu_sc as plsc


@functools.lru_cache(maxsize=None)
def _build_sc_kernel(B, H, W, C, P):
    info = plsc.get_sparse_core_info()
    NC, NS, L = info.num_cores, info.num_subcores, info.num_lanes
    NW = NC * NS
    R = B * P
    rpw = R // NW
    G = 64
    assert R % NW == 0 and P % rpw == 0 and rpw % G == 0 and C % L == 0
    n_chunks = rpw // G

    mesh = plsc.VectorSubcoreMesh(core_axis_name="c", subcore_axis_name="s")

    @functools.partial(
        pl.kernel,
        mesh=mesh,
        out_type=jax.ShapeDtypeStruct((R, 128), jnp.float32),
        compiler_params=pltpu.CompilerParams(
            needs_layout_passes=False, use_tc_tiling_on_sc=True
        ),
        scratch_types=[
            pltpu.VMEM((rpw,), jnp.int32),
            pltpu.VMEM((rpw,), jnp.int32),
            pltpu.VMEM((rpw,), jnp.float32),
            pltpu.VMEM((G, 8, C), jnp.float32),
            pltpu.VMEM((G, 128), jnp.float32),
            pltpu.SemaphoreType.DMA,
        ],
    )
    def sc_kernel(emb, yy, xx, mm, out, y_v, x_v, m_v, slab_v, out_v, sem):
        wid = lax.axis_index("s") * NC + lax.axis_index("c")
        base = wid * rpw
        pltpu.sync_copy(yy.at[pl.ds(base, rpw)], y_v)
        pltpu.sync_copy(xx.at[pl.ds(base, rpw)], x_v)
        pltpu.sync_copy(mm.at[pl.ds(base, rpw)], m_v)

        b_idx = base // P
        iota = lax.iota(jnp.int32, L)

        def chunk_body(j, carry):
            copies = []
            for g in range(G // L):
                i0 = j * G + g * L
                yv = y_v[pl.ds(i0, L)]
                x8v = x_v[pl.ds(i0, L)] & -8
                for l in range(L):
                    copies.append(pltpu.async_copy(
                        emb.at[b_idx, yv[l],
                               pl.ds(pl.multiple_of(x8v[l], 8), 8)],
                        slab_v.at[g * L + l], sem))
            for c in copies:
                c.wait()

            for g in range(G // L):
                i0 = j * G + g * L
                sv = x_v[pl.ds(i0, L)] & 7
                m16 = m_v[pl.ds(i0, L)]
                slot = iota + (g * L)
                for c in range(C):
                    cvec = jnp.full((L,), c, jnp.int32)
                    v = plsc.load_gather(slab_v, [slot, sv, cvec])
                    plsc.store_scatter(out_v, [slot, cvec], v * m16)
            pltpu.sync_copy(out_v, out.at[pl.ds(base + j * G, G)])
            return carry

        lax.fori_loop(0, n_chunks, chunk_body, 0)

    return sc_kernel


def kernel(embeddings, coords, mask):
    B, H, W, C = embeddings.shape
    P = coords.shape[1]
    c32 = coords.astype(jnp.int32)
    yy = c32[..., 0].reshape(-1)
    xx = c32[..., 1].reshape(-1)
    mm = mask.reshape(-1)
    out = _build_sc_kernel(B, H, W, C, P)(embeddings, yy, xx, mm)
    return out[:, :C].reshape(B, P, C)
